# baseline (speedup 1.0000x reference)
import jax
import jax.numpy as jnp
from jax import lax
from jax.experimental import pallas as pl
from jax.experimental.pallas import tpu as pltpu
from jax.experimental.pallas import tpu_sc as plsc

NC = 2
NS = 16
LANE = 128


def _gate_prep_body(h_ref, c_ref, W_ref, b_ref, g_ref):
    hU = lax.dot_general(h_ref[...], W_ref[...], (((1,), (1,)), ((), ())),
                         preferred_element_type=jnp.float32)
    g_ref[...] = jax.nn.sigmoid(hU + b_ref[...]) * c_ref[...]


def _xw_body(x_ref, W_ref, b_ref, o_ref):
    o_ref[...] = lax.dot_general(x_ref[...], W_ref[...],
                                 (((1,), (1,)), ((), ())),
                                 preferred_element_type=jnp.float32) + b_ref[...]


def _xw(x_p, W_iou, b_iou, blk=512):
    n_pad, H = x_p.shape
    return pl.pallas_call(
        _xw_body,
        grid=(n_pad // blk,),
        in_specs=[pl.BlockSpec((blk, H), lambda i: (i, 0)),
                  pl.BlockSpec((3 * H, H), lambda i: (0, 0)),
                  pl.BlockSpec((1, 3 * H), lambda i: (0, 0))],
        out_specs=pl.BlockSpec((blk, 3 * H), lambda i: (i, 0)),
        out_shape=jax.ShapeDtypeStruct((n_pad, 3 * H), jnp.float32),
    )(x_p, W_iou, b_iou)


def _apply_body(xw_ref, ht_ref, cred_ref, U_ref, h_out, c_out):
    iou = (xw_ref[...]
           + lax.dot_general(ht_ref[...], U_ref[...], (((1,), (1,)), ((), ())),
                             preferred_element_type=jnp.float32))
    H = ht_ref.shape[1]
    i = jax.nn.sigmoid(iou[:, :H])
    o = jax.nn.sigmoid(iou[:, H:2 * H])
    u = jnp.tanh(iou[:, 2 * H:])
    c_new = i * u + cred_ref[...]
    h_out[...] = o * jnp.tanh(c_new)
    c_out[...] = c_new


def _gate_prep(h_p, c_p, W, b, blk=512):
    n_pad, H = h_p.shape
    return pl.pallas_call(
        _gate_prep_body,
        grid=(n_pad // blk,),
        in_specs=[pl.BlockSpec((blk, H), lambda i: (i, 0)),
                  pl.BlockSpec((blk, H), lambda i: (i, 0)),
                  pl.BlockSpec((H, H), lambda i: (0, 0)),
                  pl.BlockSpec((1, H), lambda i: (0, 0))],
        out_specs=pl.BlockSpec((blk, H), lambda i: (i, 0)),
        out_shape=jax.ShapeDtypeStruct((n_pad, H), jnp.float32),
    )(h_p, c_p, W, b)


def _apply(xw, ht, cred, U_iou, blk=512):
    n_pad, H = ht.shape
    return pl.pallas_call(
        _apply_body,
        grid=(n_pad // blk,),
        in_specs=[pl.BlockSpec((blk, 3 * H), lambda i: (i, 0)),
                  pl.BlockSpec((blk, H), lambda i: (i, 0)),
                  pl.BlockSpec((blk, H), lambda i: (i, 0)),
                  pl.BlockSpec((3 * H, H), lambda i: (0, 0))],
        out_specs=[pl.BlockSpec((blk, H), lambda i: (i, 0)),
                   pl.BlockSpec((blk, H), lambda i: (i, 0))],
        out_shape=[jax.ShapeDtypeStruct((n_pad, H), jnp.float32),
                   jax.ShapeDtypeStruct((n_pad, H), jnp.float32)],
    )(xw, ht, cred, U_iou)


def _make_edge_scatter(n_pad, nb, hw):
    mesh = plsc.VectorSubcoreMesh(core_axis_name="c", subcore_axis_name="s")
    rps = n_pad // NS

    def body(t0, t1, t2, t3, srci, dsti, zeros_hbm, out,
             idx_s, idx_d, rows, acc, sem):
        cid = lax.axis_index("c")
        sid = lax.axis_index("s")
        pltpu.sync_copy(srci.at[sid], idx_s)
        pltpu.sync_copy(dsti.at[sid], idx_d)
        tables = (t0, t1, t2, t3)
        for k in range(4):
            @pl.when(cid == k // 2)
            def _chunk(tbl=tables[k], k=k):
                pltpu.sync_copy(zeros_hbm.at[pl.ds(sid * rps, rps)],
                                acc.at[pl.ds(sid * rps, rps)])
                plsc.subcore_barrier()

                def step(j, carry):
                    pltpu.async_copy(tbl.at[idx_s.at[j]], rows, sem).wait()
                    pltpu.sync_copy(rows, acc.at[idx_d.at[j]], add=True)
                    return carry

                lax.fori_loop(0, nb, step, 0)
                plsc.subcore_barrier()
                pltpu.sync_copy(acc.at[pl.ds(sid * rps, rps)],
                                out.at[k, pl.ds(sid * rps, rps)])

    return pl.kernel(
        body,
        out_type=jax.ShapeDtypeStruct((4, n_pad, hw), jnp.float32),
        mesh=mesh,
        scratch_types=[
            pltpu.VMEM((nb, LANE), jnp.int32),
            pltpu.VMEM((nb, LANE), jnp.int32),
            pltpu.VMEM((LANE, hw), jnp.float32),
            pltpu.VMEM_SHARED((n_pad, hw), jnp.float32),
            pltpu.SemaphoreType.DMA,
        ],
    )


def kernel(x, h, c, edge_index, W_iou, U_iou, b_iou, U_f_W, U_f_b):
    N, H = h.shape
    E = edge_index.shape[1]
    f32 = jnp.float32
    blk = 512
    n_pad = -(-N // blk) * blk
    hw = H // 2
    eb = NS * LANE
    nb = -(-E // eb)
    ep = nb * eb

    h_p = jnp.zeros((n_pad, H), f32).at[:N].set(h)
    c_p = jnp.zeros((n_pad, H), f32).at[:N].set(c)
    x_p = jnp.zeros((n_pad, H), f32).at[:N].set(x)
    pad = ep - E
    src = jnp.concatenate([edge_index[0], jnp.full((pad,), N, jnp.int32)])
    dst = jnp.concatenate([edge_index[1], jnp.full((pad,), N, jnp.int32)])
    srcp = src.reshape(NS, nb, LANE)
    dstp = dst.reshape(NS, nb, LANE)

    G = _gate_prep(h_p, c_p, U_f_W, U_f_b.reshape(1, H), blk)

    t0 = h_p[:, :hw]
    t1 = h_p[:, hw:]
    t2 = G[:, :hw]
    t3 = G[:, hw:]
    zeros = jnp.zeros((n_pad, hw), f32)
    seg = _make_edge_scatter(n_pad, nb, hw)(t0, t1, t2, t3, srcp, dstp, zeros)
    xw = _xw(x_p, W_iou, b_iou.reshape(1, 3 * H), blk)
    h_tild = jnp.concatenate([seg[0], seg[1]], axis=1)
    c_red = jnp.concatenate([seg[2], seg[3]], axis=1)

    h_new, c_new = _apply(xw, h_tild, c_red, U_iou, blk)
    return h_new[:N], c_new[:N]


# speedup vs baseline: 1.0253x; 1.0253x over previous
import jax
import jax.numpy as jnp
from jax import lax
from jax.experimental import pallas as pl
from jax.experimental.pallas import tpu as pltpu
from jax.experimental.pallas import tpu_sc as plsc

NC = 2
NS = 16
LANE = 128


def _gate_prep_body(h_ref, c_ref, W_ref, b_ref, g_ref):
    hU = lax.dot_general(h_ref[...], W_ref[...], (((1,), (1,)), ((), ())),
                         preferred_element_type=jnp.float32)
    g_ref[...] = jax.nn.sigmoid(hU + b_ref[...]) * c_ref[...]


def _apply_body(x_ref, ht_ref, cred_ref, W_ref, U_ref, b_ref, h_out, c_out):
    iou = (lax.dot_general(x_ref[...], W_ref[...], (((1,), (1,)), ((), ())),
                           preferred_element_type=jnp.float32)
           + lax.dot_general(ht_ref[...], U_ref[...], (((1,), (1,)), ((), ())),
                             preferred_element_type=jnp.float32)
           + b_ref[...])
    H = x_ref.shape[1]
    i = jax.nn.sigmoid(iou[:, :H])
    o = jax.nn.sigmoid(iou[:, H:2 * H])
    u = jnp.tanh(iou[:, 2 * H:])
    c_new = i * u + cred_ref[...]
    h_out[...] = o * jnp.tanh(c_new)
    c_out[...] = c_new


def _gate_prep(h_p, c_p, W, b, blk=512):
    n_pad, H = h_p.shape
    return pl.pallas_call(
        _gate_prep_body,
        grid=(n_pad // blk,),
        in_specs=[pl.BlockSpec((blk, H), lambda i: (i, 0)),
                  pl.BlockSpec((blk, H), lambda i: (i, 0)),
                  pl.BlockSpec((H, H), lambda i: (0, 0)),
                  pl.BlockSpec((1, H), lambda i: (0, 0))],
        out_specs=pl.BlockSpec((blk, H), lambda i: (i, 0)),
        out_shape=jax.ShapeDtypeStruct((n_pad, H), jnp.float32),
    )(h_p, c_p, W, b)


def _apply(x_p, ht, cred, W_iou, U_iou, b_iou, blk=512):
    n_pad, H = x_p.shape
    return pl.pallas_call(
        _apply_body,
        grid=(n_pad // blk,),
        in_specs=[pl.BlockSpec((blk, H), lambda i: (i, 0)),
                  pl.BlockSpec((blk, H), lambda i: (i, 0)),
                  pl.BlockSpec((blk, H), lambda i: (i, 0)),
                  pl.BlockSpec((3 * H, H), lambda i: (0, 0)),
                  pl.BlockSpec((3 * H, H), lambda i: (0, 0)),
                  pl.BlockSpec((1, 3 * H), lambda i: (0, 0))],
        out_specs=[pl.BlockSpec((blk, H), lambda i: (i, 0)),
                   pl.BlockSpec((blk, H), lambda i: (i, 0))],
        out_shape=[jax.ShapeDtypeStruct((n_pad, H), jnp.float32),
                   jax.ShapeDtypeStruct((n_pad, H), jnp.float32)],
    )(x_p, ht, cred, W_iou, U_iou, b_iou)


def _make_edge_scatter(n_pad, nb, hw):
    mesh = plsc.VectorSubcoreMesh(core_axis_name="c", subcore_axis_name="s")
    rps = n_pad // NS

    def body(t0, t1, t2, t3, srci, dsti, zeros_hbm, out,
             idx_s, idx_d, rows, acc, sem):
        cid = lax.axis_index("c")
        sid = lax.axis_index("s")
        pltpu.sync_copy(srci.at[sid], idx_s)
        pltpu.sync_copy(dsti.at[sid], idx_d)
        tables = (t0, t1, t2, t3)
        for k in range(4):
            @pl.when(cid == k // 2)
            def _chunk(tbl=tables[k], k=k):
                pltpu.sync_copy(zeros_hbm.at[pl.ds(sid * rps, rps)],
                                acc.at[pl.ds(sid * rps, rps)])
                plsc.subcore_barrier()

                def step(j, carry):
                    pltpu.async_copy(tbl.at[idx_s.at[j]], rows, sem).wait()
                    pltpu.sync_copy(rows, acc.at[idx_d.at[j]], add=True)
                    return carry

                lax.fori_loop(0, nb, step, 0)
                plsc.subcore_barrier()
                pltpu.sync_copy(acc.at[pl.ds(sid * rps, rps)],
                                out.at[k, pl.ds(sid * rps, rps)])

    return pl.kernel(
        body,
        out_type=jax.ShapeDtypeStruct((4, n_pad, hw), jnp.float32),
        mesh=mesh,
        scratch_types=[
            pltpu.VMEM((nb, LANE), jnp.int32),
            pltpu.VMEM((nb, LANE), jnp.int32),
            pltpu.VMEM((LANE, hw), jnp.float32),
            pltpu.VMEM_SHARED((n_pad, hw), jnp.float32),
            pltpu.SemaphoreType.DMA,
        ],
    )


def kernel(x, h, c, edge_index, W_iou, U_iou, b_iou, U_f_W, U_f_b):
    N, H = h.shape
    E = edge_index.shape[1]
    f32 = jnp.float32
    blk = 1024
    n_pad = -(-N // blk) * blk
    hw = H // 2
    eb = NS * LANE
    nb = -(-E // eb)
    ep = nb * eb

    h_p = jnp.zeros((n_pad, H), f32).at[:N].set(h)
    c_p = jnp.zeros((n_pad, H), f32).at[:N].set(c)
    x_p = jnp.zeros((n_pad, H), f32).at[:N].set(x)
    pad = ep - E
    src = jnp.concatenate([edge_index[0], jnp.full((pad,), N, jnp.int32)])
    dst = jnp.concatenate([edge_index[1], jnp.full((pad,), N, jnp.int32)])
    srcp = src.reshape(NS, nb, LANE)
    dstp = dst.reshape(NS, nb, LANE)

    G = _gate_prep(h_p, c_p, U_f_W, U_f_b.reshape(1, H), blk)

    t0 = h_p[:, :hw]
    t1 = h_p[:, hw:]
    t2 = G[:, :hw]
    t3 = G[:, hw:]
    zeros = jnp.zeros((n_pad, hw), f32)
    seg = _make_edge_scatter(n_pad, nb, hw)(t0, t1, t2, t3, srcp, dstp, zeros)
    h_tild = jnp.concatenate([seg[0], seg[1]], axis=1)
    c_red = jnp.concatenate([seg[2], seg[3]], axis=1)

    h_new, c_new = _apply(x_p, h_tild, c_red, W_iou, U_iou,
                          b_iou.reshape(1, 3 * H), blk)
    return h_new[:N], c_new[:N]


# speedup vs baseline: 1.0407x; 1.0150x over previous
import jax
import jax.numpy as jnp
from jax import lax
from jax.experimental import pallas as pl
from jax.experimental.pallas import tpu as pltpu
from jax.experimental.pallas import tpu_sc as plsc

NC = 2
NS = 16
LANE = 128


def _gate_prep_body(h_ref, c_ref, W_ref, b_ref, g_ref):
    hU = lax.dot_general(h_ref[...], W_ref[...], (((1,), (1,)), ((), ())),
                         preferred_element_type=jnp.float32)
    g_ref[...] = jax.nn.sigmoid(hU + b_ref[...]) * c_ref[...]


def _apply_body(x_ref, ht_ref, cred_ref, W_ref, U_ref, b_ref, h_out, c_out):
    iou = (lax.dot_general(x_ref[...], W_ref[...], (((1,), (1,)), ((), ())),
                           preferred_element_type=jnp.float32)
           + lax.dot_general(ht_ref[...], U_ref[...], (((1,), (1,)), ((), ())),
                             preferred_element_type=jnp.float32)
           + b_ref[...])
    H = x_ref.shape[1]
    i = jax.nn.sigmoid(iou[:, :H])
    o = jax.nn.sigmoid(iou[:, H:2 * H])
    u = jnp.tanh(iou[:, 2 * H:])
    c_new = i * u + cred_ref[...]
    h_out[...] = o * jnp.tanh(c_new)
    c_out[...] = c_new


def _gate_prep(h_p, c_p, W, b, blk=512):
    n_pad, H = h_p.shape
    return pl.pallas_call(
        _gate_prep_body,
        grid=(n_pad // blk,),
        in_specs=[pl.BlockSpec((blk, H), lambda i: (i, 0)),
                  pl.BlockSpec((blk, H), lambda i: (i, 0)),
                  pl.BlockSpec((H, H), lambda i: (0, 0)),
                  pl.BlockSpec((1, H), lambda i: (0, 0))],
        out_specs=pl.BlockSpec((blk, H), lambda i: (i, 0)),
        out_shape=jax.ShapeDtypeStruct((n_pad, H), jnp.float32),
    )(h_p, c_p, W, b)


def _apply(x_p, ht, cred, W_iou, U_iou, b_iou, blk=512):
    n_pad, H = x_p.shape
    return pl.pallas_call(
        _apply_body,
        grid=(n_pad // blk,),
        in_specs=[pl.BlockSpec((blk, H), lambda i: (i, 0)),
                  pl.BlockSpec((blk, H), lambda i: (i, 0)),
                  pl.BlockSpec((blk, H), lambda i: (i, 0)),
                  pl.BlockSpec((3 * H, H), lambda i: (0, 0)),
                  pl.BlockSpec((3 * H, H), lambda i: (0, 0)),
                  pl.BlockSpec((1, 3 * H), lambda i: (0, 0))],
        out_specs=[pl.BlockSpec((blk, H), lambda i: (i, 0)),
                   pl.BlockSpec((blk, H), lambda i: (i, 0))],
        out_shape=[jax.ShapeDtypeStruct((n_pad, H), jnp.float32),
                   jax.ShapeDtypeStruct((n_pad, H), jnp.float32)],
    )(x_p, ht, cred, W_iou, U_iou, b_iou)


def _make_edge_scatter(n_pad, nb, hw):
    mesh = plsc.VectorSubcoreMesh(core_axis_name="c", subcore_axis_name="s")
    rps = n_pad // NS

    def body(t0, t1, t2, t3, srci, dsti, zeros_hbm, out,
             idx_s, idx_d, rows, acc, sem):
        cid = lax.axis_index("c")
        sid = lax.axis_index("s")
        pltpu.sync_copy(srci.at[sid], idx_s)
        pltpu.sync_copy(dsti.at[sid], idx_d)
        tables = (t0, t1, t2, t3)
        for k in range(4):
            @pl.when(cid == k // 2)
            def _chunk(tbl=tables[k], k=k):
                pltpu.sync_copy(zeros_hbm.at[pl.ds(sid * rps, rps)],
                                acc.at[pl.ds(sid * rps, rps)])
                plsc.subcore_barrier()

                def step(j, carry):
                    pltpu.async_copy(tbl.at[idx_s.at[j]], rows, sem).wait()
                    pltpu.sync_copy(rows, acc.at[idx_d.at[j]], add=True)
                    return carry

                lax.fori_loop(0, nb, step, 0)
                plsc.subcore_barrier()
                pltpu.sync_copy(acc.at[pl.ds(sid * rps, rps)],
                                out.at[k, pl.ds(sid * rps, rps)])

    return pl.kernel(
        body,
        out_type=jax.ShapeDtypeStruct((4, n_pad, hw), jnp.float32),
        mesh=mesh,
        scratch_types=[
            pltpu.VMEM((nb, LANE), jnp.int32),
            pltpu.VMEM((nb, LANE), jnp.int32),
            pltpu.VMEM((LANE, hw), jnp.float32),
            pltpu.VMEM_SHARED((n_pad, hw), jnp.float32),
            pltpu.SemaphoreType.DMA,
        ],
    )


def kernel(x, h, c, edge_index, W_iou, U_iou, b_iou, U_f_W, U_f_b):
    N, H = h.shape
    E = edge_index.shape[1]
    f32 = jnp.float32
    blk = 2048
    n_pad = -(-N // blk) * blk
    hw = H // 2
    eb = NS * LANE
    nb = -(-E // eb)
    ep = nb * eb

    h_p = jnp.zeros((n_pad, H), f32).at[:N].set(h)
    c_p = jnp.zeros((n_pad, H), f32).at[:N].set(c)
    x_p = jnp.zeros((n_pad, H), f32).at[:N].set(x)
    pad = ep - E
    src = jnp.concatenate([edge_index[0], jnp.full((pad,), N, jnp.int32)])
    dst = jnp.concatenate([edge_index[1], jnp.full((pad,), N, jnp.int32)])
    srcp = src.reshape(NS, nb, LANE)
    dstp = dst.reshape(NS, nb, LANE)

    G = _gate_prep(h_p, c_p, U_f_W, U_f_b.reshape(1, H), blk)

    t0 = h_p[:, :hw]
    t1 = h_p[:, hw:]
    t2 = G[:, :hw]
    t3 = G[:, hw:]
    zeros = jnp.zeros((n_pad, hw), f32)
    seg = _make_edge_scatter(n_pad, nb, hw)(t0, t1, t2, t3, srcp, dstp, zeros)
    h_tild = jnp.concatenate([seg[0], seg[1]], axis=1)
    c_red = jnp.concatenate([seg[2], seg[3]], axis=1)

    h_new, c_new = _apply(x_p, h_tild, c_red, W_iou, U_iou,
                          b_iou.reshape(1, 3 * H), blk)
    return h_new[:N], c_new[:N]
